# baseline (speedup 1.0000x reference)
import jax
import jax.numpy as jnp
from jax import lax
from jax.experimental import pallas as pl
from jax.experimental.pallas import tpu as pltpu
from jax.experimental.pallas import tpu_sc as plsc

N = 10000
E = 320000
D = 128
NC = 2
NS = 16
NW = NC * NS
EPW = E // NW
CH = 80
NCHUNK = EPW // CH
RB = 1000
GRID = N // RB
TPR8 = 624
ZCH = 640

NBUF = 4
BCH = 5
NBLK = NCHUNK // BCH
IBS = 3


def _sc_edge_body(src_hbm, dst_hbm, ssrc_hbm, sdst_hbm, h_hbm,
                  acc0_hbm, acc1_hbm, den0_hbm, den1_hbm,
                  sidx, didx, wbuf, sbs, sbd, zbuf,
                  r0, r1, r2, r3,
                  acc_sh, den_sh, ssrc_sh, sdst_sh,
                  gs0, gs1, gs2, gs3,
                  qs0, qs1, qs2, qs3,
                  qd0, qd1, qd2, qd3,
                  ss0, ss1, ss2, ss3,
                  ds0, ds1, ds2, ds3):
    cid = lax.axis_index("c")
    sid = lax.axis_index("s")
    wid = cid * NS + sid
    rows = (r0, r1, r2, r3)
    gsem = (gs0, gs1, gs2, gs3)
    qssem = (qs0, qs1, qs2, qs3)
    qdsem = (qd0, qd1, qd2, qd3)
    ssem = (ss0, ss1, ss2, ss3)
    dsem = (ds0, ds1, ds2, ds3)

    def zrow_body(i, _):
        for v in range(D // 16):
            r0[i, pl.ds(v * 16, 16)] = jnp.zeros((16,), jnp.float32)
        return 0
    lax.fori_loop(0, CH, zrow_body, 0)

    def zs_body(i, _):
        zbuf[pl.ds(i * 16, 16)] = jnp.zeros((16,), jnp.float32)
        return 0
    lax.fori_loop(0, ZCH // 16, zs_body, 0)

    base_r = sid * TPR8
    for k in range(TPR8 // CH):
        pltpu.sync_copy(r0, acc_sh.at[pl.ds(base_r + k * CH, CH)])
    rem = TPR8 - (TPR8 // CH) * CH
    pltpu.sync_copy(r0.at[pl.ds(0, rem)],
                    acc_sh.at[pl.ds(base_r + (TPR8 // CH) * CH, rem)])

    @pl.when(sid == NS - 1)
    def _():
        pltpu.sync_copy(r0.at[pl.ds(0, N - NS * TPR8)],
                        acc_sh.at[pl.ds(NS * TPR8, N - NS * TPR8)])
    pltpu.sync_copy(zbuf, den_sh.at[pl.ds(sid * ZCH, ZCH)])

    @pl.when(sid == 0)
    def _():
        pltpu.sync_copy(ssrc_hbm, ssrc_sh)
        pltpu.sync_copy(sdst_hbm, sdst_sh)

    for blk in range(2):
        pltpu.sync_copy(src_hbm.at[wid, blk],
                        sidx.at[pl.ds(blk * BCH, BCH)])
        pltpu.sync_copy(dst_hbm.at[wid, blk],
                        didx.at[pl.ds(blk * BCH, BCH)])

    plsc.subcore_barrier()

    def _row_of(ci):
        blk = ci // BCH
        return (blk % IBS) * BCH + ci % BCH

    def _start_gathers(ci, b):
        row = _row_of(ci)
        pltpu.make_async_copy(h_hbm.at[sidx.at[row]], rows[b],
                              gsem[b]).start()
        pltpu.make_async_copy(ssrc_sh.at[sidx.at[row]], sbs.at[b],
                              qssem[b]).start()
        pltpu.make_async_copy(sdst_sh.at[didx.at[row]], sbd.at[b],
                              qdsem[b]).start()

    for g in range(2):
        _start_gathers(g, g)

    def _visit(ci, b):
        row = _row_of(ci)
        pltpu.make_async_copy(h_hbm.at[sidx.at[row]], rows[b],
                              gsem[b]).wait()
        pltpu.make_async_copy(ssrc_sh.at[sidx.at[row]], sbs.at[b],
                              qssem[b]).wait()
        pltpu.make_async_copy(sdst_sh.at[didx.at[row]], sbd.at[b],
                              qdsem[b]).wait()

        blk = ci // BCH

        @pl.when((ci % BCH == 0) & (blk >= 1) & (blk < NBLK - 1))
        def _():
            p3 = ((blk + 1) % IBS) * BCH
            pltpu.sync_copy(src_hbm.at[wid, blk + 1],
                            sidx.at[pl.ds(p3, BCH)])
            pltpu.sync_copy(dst_hbm.at[wid, blk + 1],
                            didx.at[pl.ds(p3, BCH)])

        @plsc.parallel_loop(0, CH // 16, 1)
        def _scores(g):
            e = sbs[b, pl.ds(g * 16, 16)] + sbd[b, pl.ds(g * 16, 16)]
            e = jnp.where(e >= 0.0, e, 0.2 * e)
            wbuf[b, pl.ds(g * 16, 16)] = jnp.exp(e)

        @plsc.parallel_loop(0, CH // 16, 1, unroll=2)
        def _scale(g):
            ex = wbuf[b, pl.ds(g * 16, 16)]
            for l in range(16):
                w = ex[l]
                r = g * 16 + l
                for v in range(D // 16):
                    rows[b][r, pl.ds(v * 16, 16)] = \
                        rows[b][r, pl.ds(v * 16, 16)] * w

        pltpu.sync_copy(rows[b], acc_sh.at[didx.at[row]], add=True)
        pltpu.sync_copy(wbuf.at[b], den_sh.at[didx.at[row]], add=True)

        nx = ci + 2
        bg = (b + 2) % NBUF

        @pl.when(nx < NCHUNK)
        def _():
            _start_gathers(nx, bg)

    def chunk_body(k, _):
        for b in range(NBUF):
            _visit(k * NBUF + b, b)
        return 0
    lax.fori_loop(0, NCHUNK // NBUF, chunk_body, 0)
    for t in range(NCHUNK - (NCHUNK // NBUF) * NBUF):
        ci = (NCHUNK // NBUF) * NBUF + t
        _visit(jnp.int32(ci), ci % NBUF)

    plsc.subcore_barrier()

    TAIL = N - NS * TPR8

    @pl.when(cid == 0)
    def _():
        pltpu.sync_copy(acc_sh.at[pl.ds(sid * TPR8, TPR8)],
                        acc0_hbm.at[pl.ds(sid * TPR8, TPR8)])

    @pl.when(cid == 1)
    def _():
        pltpu.sync_copy(acc_sh.at[pl.ds(sid * TPR8, TPR8)],
                        acc1_hbm.at[pl.ds(sid * TPR8, TPR8)])

    @pl.when((cid == 0) & (sid == NS - 1))
    def _():
        pltpu.sync_copy(acc_sh.at[pl.ds(NS * TPR8, TAIL)],
                        acc0_hbm.at[pl.ds(NS * TPR8, TAIL)])

    @pl.when((cid == 1) & (sid == NS - 1))
    def _():
        pltpu.sync_copy(acc_sh.at[pl.ds(NS * TPR8, TAIL)],
                        acc1_hbm.at[pl.ds(NS * TPR8, TAIL)])

    @pl.when((cid == 0) & (sid == 0))
    def _():
        pltpu.sync_copy(den_sh.at[pl.ds(0, N)], den0_hbm)

    @pl.when((cid == 1) & (sid == 0))
    def _():
        pltpu.sync_copy(den_sh.at[pl.ds(0, N)], den1_hbm)


_SC_EDGE = pl.kernel(
    _sc_edge_body,
    out_type=[jax.ShapeDtypeStruct((N, D), jnp.float32),
              jax.ShapeDtypeStruct((N, D), jnp.float32),
              jax.ShapeDtypeStruct((N,), jnp.float32),
              jax.ShapeDtypeStruct((N,), jnp.float32)],
    mesh=plsc.VectorSubcoreMesh(core_axis_name="c", subcore_axis_name="s",
                                num_cores=NC, num_subcores=NS),
    compiler_params=pltpu.CompilerParams(use_tc_tiling_on_sc=False,
                                         needs_layout_passes=False),
    scratch_types=(
        [pltpu.VMEM((IBS * BCH, CH), jnp.int32),
         pltpu.VMEM((IBS * BCH, CH), jnp.int32),
         pltpu.VMEM((NBUF, CH), jnp.float32),
         pltpu.VMEM((NBUF, CH), jnp.float32),
         pltpu.VMEM((NBUF, CH), jnp.float32),
         pltpu.VMEM((ZCH,), jnp.float32)]
        + [pltpu.VMEM((CH, D), jnp.float32) for _ in range(NBUF)]
        + [pltpu.VMEM_SHARED((N, D), jnp.float32),
           pltpu.VMEM_SHARED((NS * ZCH,), jnp.float32),
           pltpu.VMEM_SHARED((N,), jnp.float32),
           pltpu.VMEM_SHARED((N,), jnp.float32)]
        + [pltpu.SemaphoreType.DMA] * (5 * NBUF)
    ),
)


def _tc_front_body(x_ref, w_ref, a_ref, h_ref, s_ref):
    h = jnp.dot(x_ref[...], w_ref[...], preferred_element_type=jnp.float32)
    h_ref[...] = h
    s_ref[...] = jnp.dot(h, a_ref[...], preferred_element_type=jnp.float32)


_TC_FRONT = pl.pallas_call(
    _tc_front_body,
    grid=(GRID,),
    in_specs=[pl.BlockSpec((RB, D), lambda i: (i, 0)),
              pl.BlockSpec((D, D), lambda i: (0, 0)),
              pl.BlockSpec((D, 2), lambda i: (0, 0))],
    out_specs=[pl.BlockSpec((RB, D), lambda i: (i, 0)),
               pl.BlockSpec((RB, 2), lambda i: (i, 0))],
    out_shape=[jax.ShapeDtypeStruct((N, D), jnp.float32),
               jax.ShapeDtypeStruct((N, 2), jnp.float32)],
)


def _tc_mid_body(a0_ref, a1_ref, d0_ref, d1_ref, b_ref, w_ref, a_ref,
                 h_ref, s_ref):
    den = d0_ref[...] + d1_ref[...] + 1e-16
    hin = (a0_ref[...] + a1_ref[...]) / den + b_ref[...]
    hin = jnp.maximum(hin, 0.0)
    h = jnp.dot(hin, w_ref[...], preferred_element_type=jnp.float32)
    h_ref[...] = h
    s_ref[...] = jnp.dot(h, a_ref[...], preferred_element_type=jnp.float32)


_TC_MID = pl.pallas_call(
    _tc_mid_body,
    grid=(GRID,),
    in_specs=[pl.BlockSpec((RB, D), lambda i: (i, 0)),
              pl.BlockSpec((RB, D), lambda i: (i, 0)),
              pl.BlockSpec((RB, 1), lambda i: (i, 0)),
              pl.BlockSpec((RB, 1), lambda i: (i, 0)),
              pl.BlockSpec((1, D), lambda i: (0, 0)),
              pl.BlockSpec((D, D), lambda i: (0, 0)),
              pl.BlockSpec((D, 2), lambda i: (0, 0))],
    out_specs=[pl.BlockSpec((RB, D), lambda i: (i, 0)),
               pl.BlockSpec((RB, 2), lambda i: (i, 0))],
    out_shape=[jax.ShapeDtypeStruct((N, D), jnp.float32),
               jax.ShapeDtypeStruct((N, 2), jnp.float32)],
)


def _tc_final_body(a0_ref, a1_ref, d0_ref, d1_ref, b_ref, out_ref):
    den = d0_ref[...] + d1_ref[...] + 1e-16
    out_ref[...] = (a0_ref[...] + a1_ref[...]) / den + b_ref[...]


_TC_FINAL = pl.pallas_call(
    _tc_final_body,
    grid=(GRID,),
    in_specs=[pl.BlockSpec((RB, D), lambda i: (i, 0)),
              pl.BlockSpec((RB, D), lambda i: (i, 0)),
              pl.BlockSpec((RB, 1), lambda i: (i, 0)),
              pl.BlockSpec((RB, 1), lambda i: (i, 0)),
              pl.BlockSpec((1, D), lambda i: (0, 0))],
    out_specs=pl.BlockSpec((RB, D), lambda i: (i, 0)),
    out_shape=jax.ShapeDtypeStruct((N, D), jnp.float32),
)


def kernel(x, edge_index, W1, a1_src, a1_dst, b1, W2, a2_src, a2_dst, b2):
    src = edge_index[0].reshape(NW, NBLK, BCH, CH)
    dst = edge_index[1].reshape(NW, NBLK, BCH, CH)
    A1 = jnp.stack([a1_src, a1_dst], axis=1)
    A2 = jnp.stack([a2_src, a2_dst], axis=1)

    h1, s1 = _TC_FRONT(x, W1, A1)
    acc0, acc1, den0, den1 = _SC_EDGE(src, dst, s1[:, 0], s1[:, 1], h1)
    h2, s2 = _TC_MID(acc0, acc1, den0[:, None], den1[:, None],
                     b1[None, :], W2, A2)
    p0, p1, q0, q1 = _SC_EDGE(src, dst, s2[:, 0], s2[:, 1], h2)
    out = _TC_FINAL(p0, p1, q0[:, None], q1[:, None], b2[None, :])
    return out


# speedup vs baseline: 1.2610x; 1.2610x over previous
import jax
import jax.numpy as jnp
from jax import lax
from jax.experimental import pallas as pl
from jax.experimental.pallas import tpu as pltpu
from jax.experimental.pallas import tpu_sc as plsc

N = 10000
E = 320000
D = 128
NC = 2
NS = 16
NW = NC * NS
EPW = E // NW
CH = 80
NCHUNK = EPW // CH
RB = 1000
GRID = N // RB
TPR8 = 624
ZCH = 640

NBUF = 3
BCH = 5
NBLK = NCHUNK // BCH
IBS = 3


def _sc_edge_body(src_hbm, dst_hbm, ssrc_hbm, sdst_hbm, h_hbm,
                  acc0_hbm, acc1_hbm, den0_hbm, den1_hbm,
                  sidx, didx, wbuf, sbs, sdst_v, zbuf,
                  r0, r1, r2,
                  acc_sh, den_sh, ssrc_sh,
                  gs0, gs1, gs2,
                  qs0, qs1, qs2,
                  ss0, ss1, ss2,
                  ds0, ds1, ds2):
    cid = lax.axis_index("c")
    sid = lax.axis_index("s")
    wid = cid * NS + sid
    rows = (r0, r1, r2)
    gsem = (gs0, gs1, gs2)
    qssem = (qs0, qs1, qs2)
    ssem = (ss0, ss1, ss2)
    dsem = (ds0, ds1, ds2)

    def zrow_body(i, _):
        for v in range(D // 16):
            r0[i, pl.ds(v * 16, 16)] = jnp.zeros((16,), jnp.float32)
        return 0
    lax.fori_loop(0, CH, zrow_body, 0)

    def zs_body(i, _):
        zbuf[pl.ds(i * 16, 16)] = jnp.zeros((16,), jnp.float32)
        return 0
    lax.fori_loop(0, ZCH // 16, zs_body, 0)

    base_r = sid * TPR8
    for k in range(TPR8 // CH):
        pltpu.sync_copy(r0, acc_sh.at[pl.ds(base_r + k * CH, CH)])
    rem = TPR8 - (TPR8 // CH) * CH
    pltpu.sync_copy(r0.at[pl.ds(0, rem)],
                    acc_sh.at[pl.ds(base_r + (TPR8 // CH) * CH, rem)])

    @pl.when(sid == NS - 1)
    def _():
        pltpu.sync_copy(r0.at[pl.ds(0, N - NS * TPR8)],
                        acc_sh.at[pl.ds(NS * TPR8, N - NS * TPR8)])
    pltpu.sync_copy(zbuf, den_sh.at[pl.ds(sid * ZCH, ZCH)])

    @pl.when(sid == 0)
    def _():
        pltpu.sync_copy(ssrc_hbm, ssrc_sh)
    pltpu.sync_copy(sdst_hbm, sdst_v)

    for blk in range(2):
        pltpu.sync_copy(src_hbm.at[wid, blk],
                        sidx.at[pl.ds(blk * BCH, BCH)])
        pltpu.sync_copy(dst_hbm.at[wid, blk],
                        didx.at[pl.ds(blk * BCH, BCH)])

    plsc.subcore_barrier()

    def _row_of(ci):
        blk = ci // BCH
        return (blk % IBS) * BCH + ci % BCH

    def _start_gathers(ci, b):
        row = _row_of(ci)
        pltpu.make_async_copy(h_hbm.at[sidx.at[row]], rows[b],
                              gsem[b]).start()
        pltpu.make_async_copy(ssrc_sh.at[sidx.at[row]], sbs.at[b],
                              qssem[b]).start()

    for g in range(2):
        _start_gathers(g, g)

    def _visit(ci, b):
        row = _row_of(ci)
        pltpu.make_async_copy(h_hbm.at[sidx.at[row]], rows[b],
                              gsem[b]).wait()
        pltpu.make_async_copy(ssrc_sh.at[sidx.at[row]], sbs.at[b],
                              qssem[b]).wait()

        blk = ci // BCH

        @pl.when((ci % BCH == 0) & (blk >= 1) & (blk < NBLK - 1))
        def _():
            p3 = ((blk + 1) % IBS) * BCH
            pltpu.sync_copy(src_hbm.at[wid, blk + 1],
                            sidx.at[pl.ds(p3, BCH)])
            pltpu.sync_copy(dst_hbm.at[wid, blk + 1],
                            didx.at[pl.ds(p3, BCH)])

        @plsc.parallel_loop(0, CH // 16, 1)
        def _scores(g):
            di = didx[row, pl.ds(g * 16, 16)]
            e = sbs[b, pl.ds(g * 16, 16)] + plsc.load_gather(sdst_v, [di])
            e = jnp.where(e >= 0.0, e, 0.2 * e)
            wbuf[b, pl.ds(g * 16, 16)] = jnp.exp(e)

        @plsc.parallel_loop(0, CH // 16, 1, unroll=2)
        def _scale(g):
            ex = wbuf[b, pl.ds(g * 16, 16)]
            for l in range(16):
                w = ex[l]
                r = g * 16 + l
                for v in range(D // 16):
                    rows[b][r, pl.ds(v * 16, 16)] = \
                        rows[b][r, pl.ds(v * 16, 16)] * w

        pltpu.async_copy(rows[b], acc_sh.at[didx.at[row]], ssem[b],
                         add=True)
        pltpu.async_copy(wbuf.at[b], den_sh.at[didx.at[row]], dsem[b],
                         add=True)

        nx = ci + 2
        bg = (b + 2) % NBUF

        @pl.when(nx < NCHUNK)
        def _():
            @pl.when(nx >= NBUF)
            def _():
                rp = _row_of(nx - NBUF)
                pltpu.make_async_copy(rows[bg], acc_sh.at[didx.at[rp]],
                                      ssem[bg]).wait()
                pltpu.make_async_copy(wbuf.at[bg], den_sh.at[didx.at[rp]],
                                      dsem[bg]).wait()
            _start_gathers(nx, bg)

    def chunk_body(k, _):
        for b in range(NBUF):
            _visit(k * NBUF + b, b)
        return 0
    lax.fori_loop(0, NCHUNK // NBUF, chunk_body, 0)
    for t in range(NCHUNK - (NCHUNK // NBUF) * NBUF):
        ci = (NCHUNK // NBUF) * NBUF + t
        _visit(jnp.int32(ci), ci % NBUF)

    for c in range(NCHUNK - NBUF, NCHUNK):
        b = c % NBUF
        rp = _row_of(c)
        pltpu.make_async_copy(rows[b], acc_sh.at[didx.at[rp]],
                              ssem[b]).wait()
        pltpu.make_async_copy(wbuf.at[b], den_sh.at[didx.at[rp]],
                              dsem[b]).wait()

    plsc.subcore_barrier()

    TAIL = N - NS * TPR8

    @pl.when(cid == 0)
    def _():
        pltpu.sync_copy(acc_sh.at[pl.ds(sid * TPR8, TPR8)],
                        acc0_hbm.at[pl.ds(sid * TPR8, TPR8)])

    @pl.when(cid == 1)
    def _():
        pltpu.sync_copy(acc_sh.at[pl.ds(sid * TPR8, TPR8)],
                        acc1_hbm.at[pl.ds(sid * TPR8, TPR8)])

    @pl.when((cid == 0) & (sid == NS - 1))
    def _():
        pltpu.sync_copy(acc_sh.at[pl.ds(NS * TPR8, TAIL)],
                        acc0_hbm.at[pl.ds(NS * TPR8, TAIL)])

    @pl.when((cid == 1) & (sid == NS - 1))
    def _():
        pltpu.sync_copy(acc_sh.at[pl.ds(NS * TPR8, TAIL)],
                        acc1_hbm.at[pl.ds(NS * TPR8, TAIL)])

    @pl.when((cid == 0) & (sid == 0))
    def _():
        pltpu.sync_copy(den_sh.at[pl.ds(0, N)], den0_hbm)

    @pl.when((cid == 1) & (sid == 0))
    def _():
        pltpu.sync_copy(den_sh.at[pl.ds(0, N)], den1_hbm)


_SC_EDGE = pl.kernel(
    _sc_edge_body,
    out_type=[jax.ShapeDtypeStruct((N, D), jnp.float32),
              jax.ShapeDtypeStruct((N, D), jnp.float32),
              jax.ShapeDtypeStruct((N,), jnp.float32),
              jax.ShapeDtypeStruct((N,), jnp.float32)],
    mesh=plsc.VectorSubcoreMesh(core_axis_name="c", subcore_axis_name="s",
                                num_cores=NC, num_subcores=NS),
    compiler_params=pltpu.CompilerParams(use_tc_tiling_on_sc=False,
                                         needs_layout_passes=False),
    scratch_types=(
        [pltpu.VMEM((IBS * BCH, CH), jnp.int32),
         pltpu.VMEM((IBS * BCH, CH), jnp.int32),
         pltpu.VMEM((NBUF, CH), jnp.float32),
         pltpu.VMEM((NBUF, CH), jnp.float32),
         pltpu.VMEM((N,), jnp.float32),
         pltpu.VMEM((ZCH,), jnp.float32)]
        + [pltpu.VMEM((CH, D), jnp.float32) for _ in range(NBUF)]
        + [pltpu.VMEM_SHARED((N, D), jnp.float32),
           pltpu.VMEM_SHARED((NS * ZCH,), jnp.float32),
           pltpu.VMEM_SHARED((N,), jnp.float32)]
        + [pltpu.SemaphoreType.DMA] * (4 * NBUF)
    ),
)


def _tc_front_body(x_ref, w_ref, a_ref, h_ref, s_ref):
    h = jnp.dot(x_ref[...], w_ref[...], preferred_element_type=jnp.float32)
    h_ref[...] = h
    s_ref[...] = jnp.dot(h, a_ref[...], preferred_element_type=jnp.float32)


_TC_FRONT = pl.pallas_call(
    _tc_front_body,
    grid=(GRID,),
    in_specs=[pl.BlockSpec((RB, D), lambda i: (i, 0)),
              pl.BlockSpec((D, D), lambda i: (0, 0)),
              pl.BlockSpec((D, 2), lambda i: (0, 0))],
    out_specs=[pl.BlockSpec((RB, D), lambda i: (i, 0)),
               pl.BlockSpec((RB, 2), lambda i: (i, 0))],
    out_shape=[jax.ShapeDtypeStruct((N, D), jnp.float32),
               jax.ShapeDtypeStruct((N, 2), jnp.float32)],
)


def _tc_mid_body(a0_ref, a1_ref, d0_ref, d1_ref, b_ref, w_ref, a_ref,
                 h_ref, s_ref):
    den = d0_ref[...] + d1_ref[...] + 1e-16
    hin = (a0_ref[...] + a1_ref[...]) / den + b_ref[...]
    hin = jnp.maximum(hin, 0.0)
    h = jnp.dot(hin, w_ref[...], preferred_element_type=jnp.float32)
    h_ref[...] = h
    s_ref[...] = jnp.dot(h, a_ref[...], preferred_element_type=jnp.float32)


_TC_MID = pl.pallas_call(
    _tc_mid_body,
    grid=(GRID,),
    in_specs=[pl.BlockSpec((RB, D), lambda i: (i, 0)),
              pl.BlockSpec((RB, D), lambda i: (i, 0)),
              pl.BlockSpec((RB, 1), lambda i: (i, 0)),
              pl.BlockSpec((RB, 1), lambda i: (i, 0)),
              pl.BlockSpec((1, D), lambda i: (0, 0)),
              pl.BlockSpec((D, D), lambda i: (0, 0)),
              pl.BlockSpec((D, 2), lambda i: (0, 0))],
    out_specs=[pl.BlockSpec((RB, D), lambda i: (i, 0)),
               pl.BlockSpec((RB, 2), lambda i: (i, 0))],
    out_shape=[jax.ShapeDtypeStruct((N, D), jnp.float32),
               jax.ShapeDtypeStruct((N, 2), jnp.float32)],
)


def _tc_final_body(a0_ref, a1_ref, d0_ref, d1_ref, b_ref, out_ref):
    den = d0_ref[...] + d1_ref[...] + 1e-16
    out_ref[...] = (a0_ref[...] + a1_ref[...]) / den + b_ref[...]


_TC_FINAL = pl.pallas_call(
    _tc_final_body,
    grid=(GRID,),
    in_specs=[pl.BlockSpec((RB, D), lambda i: (i, 0)),
              pl.BlockSpec((RB, D), lambda i: (i, 0)),
              pl.BlockSpec((RB, 1), lambda i: (i, 0)),
              pl.BlockSpec((RB, 1), lambda i: (i, 0)),
              pl.BlockSpec((1, D), lambda i: (0, 0))],
    out_specs=pl.BlockSpec((RB, D), lambda i: (i, 0)),
    out_shape=jax.ShapeDtypeStruct((N, D), jnp.float32),
)


def kernel(x, edge_index, W1, a1_src, a1_dst, b1, W2, a2_src, a2_dst, b2):
    src = edge_index[0].reshape(NW, NBLK, BCH, CH)
    dst = edge_index[1].reshape(NW, NBLK, BCH, CH)
    A1 = jnp.stack([a1_src, a1_dst], axis=1)
    A2 = jnp.stack([a2_src, a2_dst], axis=1)

    h1, s1 = _TC_FRONT(x, W1, A1)
    acc0, acc1, den0, den1 = _SC_EDGE(src, dst, s1[:, 0], s1[:, 1], h1)
    h2, s2 = _TC_MID(acc0, acc1, den0[:, None], den1[:, None],
                     b1[None, :], W2, A2)
    p0, p1, q0, q1 = _SC_EDGE(src, dst, s2[:, 0], s2[:, 1], h2)
    out = _TC_FINAL(p0, p1, q0[:, None], q1[:, None], b2[None, :])
    return out
